# baseline (speedup 1.0000x reference)
import functools

import jax
import jax.numpy as jnp
from jax import lax
from jax.experimental import pallas as pl
from jax.experimental.pallas import tpu as pltpu
from jax.experimental.pallas import tpu_sc as plsc

_N = 10000
_E = 320000
_G = 64
_EPS = 1e-5

_NC = 2
_NS = 16
_NW = _NC * _NS
_NPAD = 10240
_EW = _E // _NW
_CH = 80
_NCHUNK = _EW // _CH
_RPT = _NPAD // _NS


def _mesh():
    return plsc.VectorSubcoreMesh(core_axis_name="c", subcore_axis_name="s",
                                  num_cores=_NC, num_subcores=_NS)


_FK = 25


@functools.lru_cache(maxsize=None)
def _make_sc_deg():
    @functools.partial(
        pl.kernel,
        out_type=jax.ShapeDtypeStruct((_NC, _NPAD), jnp.float32),
        mesh=_mesh(),
        scratch_types=[
            pltpu.VMEM((_CH,), jnp.int32),
            pltpu.VMEM((_CH,), jnp.int32),
            pltpu.VMEM((_CH,), jnp.float32),
            pltpu.VMEM_SHARED((_NPAD,), jnp.float32),
            pltpu.SemaphoreType.DMA,
            pltpu.SemaphoreType.DMA,
        ],
    )
    def deg(dsti, zeros_hbm, ones_hbm, out, didx0, didx1, ones_v, acc,
            sem0, sem1):
        didxs = (didx0, didx1)
        sems = (sem0, sem1)
        c = lax.axis_index("c")
        s = lax.axis_index("s")
        wid = s * _NC + c
        pltpu.sync_copy(zeros_hbm.at[pl.ds(0, _RPT)],
                        acc.at[pl.ds(s * _RPT, _RPT)])
        pltpu.sync_copy(ones_hbm, ones_v)
        plsc.subcore_barrier()
        ebase = wid * _EW

        def group(g, carry):
            descs = []
            for b in range(2):
                off = ebase + (g * 2 + b) * _CH
                pltpu.sync_copy(dsti.at[pl.ds(off, _CH)], didxs[b])
                descs.append(pltpu.async_copy(ones_v, acc.at[didxs[b]],
                                              sems[b], add=True))
            for b in range(2):
                descs[b].wait()
            return carry

        lax.fori_loop(0, _NCHUNK // 2, group, 0)
        off = ebase + (_NCHUNK - 1) * _CH
        pltpu.sync_copy(dsti.at[pl.ds(off, _CH)], didx0)
        pltpu.sync_copy(ones_v, acc.at[didx0], add=True)
        plsc.subcore_barrier()
        pltpu.sync_copy(acc.at[pl.ds(s * _RPT, _RPT)],
                        out.at[c, pl.ds(s * _RPT, _RPT)])

    return deg


_NB = 3
_NG = _NCHUNK // _NB
_NTAIL = _NCHUNK - _NG * _NB


@functools.lru_cache(maxsize=None)
def _make_sc_agg(D, tc_tiling=True):
    params = None if tc_tiling else pltpu.CompilerParams(
        use_tc_tiling_on_sc=False)
    @functools.partial(
        pl.kernel,
        out_type=jax.ShapeDtypeStruct((_NC, _NPAD, D), jnp.float32),
        mesh=_mesh(),
        compiler_params=params,
        scratch_types=(
            [pltpu.VMEM((_CH,), jnp.int32)] * _NB
            + [pltpu.VMEM((_CH,), jnp.int32)] * _NB
            + [pltpu.VMEM((_CH, D), jnp.float32)] * _NB
            + [pltpu.VMEM_SHARED((_NPAD, D), jnp.float32)]
            + [pltpu.SemaphoreType.DMA] * (2 * _NB)
        ),
    )
    def agg(hs, srci, dsti, zeros_hbm, out, *refs):
        sidxs = refs[:_NB]
        didxs = refs[_NB:2 * _NB]
        rowss = refs[2 * _NB:3 * _NB]
        acc = refs[3 * _NB]
        sems = refs[3 * _NB + 1:3 * _NB + 1 + _NB]
        ssems = refs[3 * _NB + 1 + _NB:]
        c = lax.axis_index("c")
        s = lax.axis_index("s")
        wid = s * _NC + c
        pltpu.sync_copy(zeros_hbm, acc.at[pl.ds(s * _RPT, _RPT)])
        plsc.subcore_barrier()
        ebase = wid * _EW

        def group(g, carry):
            descs = []
            for b in range(_NB):
                off = ebase + (g * _NB + b) * _CH
                pltpu.sync_copy(srci.at[pl.ds(off, _CH)], sidxs[b])
                pltpu.sync_copy(dsti.at[pl.ds(off, _CH)], didxs[b])
                descs.append(pltpu.async_copy(hs.at[sidxs[b]], rowss[b],
                                              sems[b]))
            sdescs = []
            for b in range(_NB):
                descs[b].wait()
                sdescs.append(pltpu.async_copy(rowss[b], acc.at[didxs[b]],
                                               ssems[b], add=True))
            for b in range(_NB):
                sdescs[b].wait()
            return carry

        lax.fori_loop(0, _NG, group, 0)
        for t in range(_NTAIL):
            off = ebase + (_NG * _NB + t) * _CH
            pltpu.sync_copy(srci.at[pl.ds(off, _CH)], sidxs[0])
            pltpu.sync_copy(dsti.at[pl.ds(off, _CH)], didxs[0])
            pltpu.async_copy(hs.at[sidxs[0]], rowss[0], sems[0]).wait()
            pltpu.sync_copy(rowss[0], acc.at[didxs[0]], add=True)
        plsc.subcore_barrier()
        pltpu.sync_copy(acc.at[pl.ds(s * _RPT, _RPT)],
                        out.at[c, pl.ds(s * _RPT, _RPT)])

    return jax.jit(agg)


def _dinv_from(degT_ref):
    deg = degT_ref[0:_N, 0:1] + degT_ref[0:_N, 1:2] + 1.0
    return lax.rsqrt(deg)


def _tc_a_body(x_ref, w1_ref, degT_ref, hs1_ref):
    dinv = _dinv_from(degT_ref)
    h = jnp.dot(x_ref[...], w1_ref[...], preferred_element_type=jnp.float32,
                precision=lax.Precision.HIGHEST)
    hs1_ref[...] = h * dinv


def _tc_b_body(agg_ref, hs1_ref, degT_ref, b1_ref, g1_ref, be1_ref, w2_ref,
               hs2_ref):
    dinv = _dinv_from(degT_ref)
    aggsum = agg_ref[0, 0:_N, :] + agg_ref[1, 0:_N, :]
    t = dinv * (aggsum + hs1_ref[...]) + b1_ref[...]
    mu = jnp.mean(t, axis=0, keepdims=True)
    var = jnp.mean((t - mu) ** 2, axis=0, keepdims=True)
    tn = g1_ref[...] * (t - mu) * lax.rsqrt(var + _EPS) + be1_ref[...]
    h = jnp.maximum(tn, 0.0)
    h2 = jnp.dot(h, w2_ref[...], preferred_element_type=jnp.float32,
                 precision=lax.Precision.HIGHEST)
    hs2_ref[...] = jnp.concatenate(
        [h2 * dinv, jnp.zeros((_N, 64), jnp.float32)], axis=1)


def _tc_c_body(agg_ref, hs2_ref, degT_ref, b2_ref, g2_ref, be2_ref,
               bcol_ref, blane_ref, G1_ref, gb1_ref, G2_ref, gb2_ref,
               C1_ref, cb1_ref, C2_ref, cb2_ref, out_ref):
    dinv = _dinv_from(degT_ref)
    aggsum = agg_ref[0, 0:_N, 0:64] + agg_ref[1, 0:_N, 0:64]
    t = dinv * (aggsum + hs2_ref[0:_N, 0:64]) + b2_ref[...]
    mu = jnp.mean(t, axis=0, keepdims=True)
    var = jnp.mean((t - mu) ** 2, axis=0, keepdims=True)
    tn = g2_ref[...] * (t - mu) * lax.rsqrt(var + _EPS) + be2_ref[...]
    h = jnp.maximum(tn, 0.0)

    g_hidden = jnp.maximum(
        jnp.dot(h, G1_ref[...], preferred_element_type=jnp.float32,
                precision=lax.Precision.HIGHEST) + gb1_ref[...], 0.0)
    gate = jnp.dot(g_hidden, G2_ref[...], preferred_element_type=jnp.float32,
                   precision=lax.Precision.HIGHEST) + gb2_ref[...]

    iota_col = lax.broadcasted_iota(jnp.int32, (_N, _G), 1)
    maskf = (bcol_ref[...] == iota_col).astype(jnp.float32)
    iota_lane = lax.broadcasted_iota(jnp.int32, (_G, _N), 0)
    maskTf = (blane_ref[...] == iota_lane).astype(jnp.float32)

    neg = jnp.float32(-jnp.inf)
    gmax = jnp.max(jnp.where(maskf > 0.0, gate, neg), axis=0, keepdims=True)
    gmax = jnp.where(jnp.isfinite(gmax), gmax, 0.0)
    gmaxn = jnp.sum(maskf * gmax, axis=1, keepdims=True)
    e = jnp.exp(gate - gmaxn)
    denom = jnp.sum(maskf * e, axis=0, keepdims=True)
    denomn = jnp.sum(maskf * denom, axis=1, keepdims=True)
    alpha = e / (denomn + 1e-16)
    weighted = alpha * h
    pooled = jnp.dot(maskTf, weighted, preferred_element_type=jnp.float32,
                     precision=lax.Precision.HIGHEST)

    z = jnp.maximum(
        jnp.dot(pooled, C1_ref[...], preferred_element_type=jnp.float32,
                precision=lax.Precision.HIGHEST) + cb1_ref[...], 0.0)
    out_ref[...] = jnp.dot(z, C2_ref[...], preferred_element_type=jnp.float32,
                           precision=lax.Precision.HIGHEST) + cb2_ref[...]


def kernel(x, edge_index, batch, W1, b1, gamma1, beta1, W2, b2, gamma2, beta2,
           G1, gb1, G2, gb2, C1, cb1, C2, cb2):
    src = edge_index[0]
    dst = edge_index[1]

    z1 = jnp.zeros((_RPT,), jnp.float32)
    ones = jnp.ones((_CH,), jnp.float32)
    degp = _make_sc_deg()(dst, z1, ones)
    degT = degp.T

    hs1 = pl.pallas_call(
        _tc_a_body,
        out_shape=jax.ShapeDtypeStruct((_N, 128), jnp.float32),
    )(x, W1, degT)

    z128 = jnp.zeros((_RPT, 128), jnp.float32)
    agg1 = _make_sc_agg(128)(hs1, src, dst, z128)

    hs2 = pl.pallas_call(
        _tc_b_body,
        out_shape=jax.ShapeDtypeStruct((_N, 128), jnp.float32),
    )(agg1, hs1, degT, b1.reshape(1, 128), gamma1.reshape(1, 128),
      beta1.reshape(1, 128), W2)

    agg2 = _make_sc_agg(128)(hs2, src, dst, z128)

    out = pl.pallas_call(
        _tc_c_body,
        out_shape=jax.ShapeDtypeStruct((_G, 2), jnp.float32),
    )(agg2, hs2, degT, b2.reshape(1, 64), gamma2.reshape(1, 64),
      beta2.reshape(1, 64), batch.reshape(_N, 1), batch.reshape(1, _N),
      G1, gb1.reshape(1, 32), G2, gb2.reshape(1, 1),
      C1, cb1.reshape(1, 32), C2, cb2.reshape(1, 2))
    return out


# speedup vs baseline: 1.1141x; 1.1141x over previous
import functools

import jax
import jax.numpy as jnp
from jax import lax
from jax.experimental import pallas as pl
from jax.experimental.pallas import tpu as pltpu
from jax.experimental.pallas import tpu_sc as plsc

_N = 10000
_E = 320000
_G = 64
_EPS = 1e-5

_NC = 2
_NS = 16
_NW = _NC * _NS
_NPAD = 10240
_EW = _E // _NW
_CH = 80
_NCHUNK = _EW // _CH
_RPT = _NPAD // _NS


def _mesh():
    return plsc.VectorSubcoreMesh(core_axis_name="c", subcore_axis_name="s",
                                  num_cores=_NC, num_subcores=_NS)


_FK = 25


@functools.lru_cache(maxsize=None)
def _make_sc_deg():
    @functools.partial(
        pl.kernel,
        out_type=jax.ShapeDtypeStruct((_NW, _NPAD), jnp.float32),
        mesh=_mesh(),
        compiler_params=pltpu.CompilerParams(needs_layout_passes=False),
        scratch_types=[
            pltpu.VMEM((_EW,), jnp.int32),
            pltpu.VMEM((_NPAD,), jnp.float32),
        ],
    )
    def deg(dsti, out, didx, hist):
        c = lax.axis_index("c")
        s = lax.axis_index("s")
        wid = s * _NC + c
        pltpu.sync_copy(dsti.at[pl.ds(wid * _EW, _EW)], didx)

        def zero(i, carry):
            hist[pl.ds(i * 16, 16)] = jnp.zeros((16,), jnp.float32)
            return carry

        lax.fori_loop(0, _NPAD // 16, zero, 0)
        ones16 = jnp.ones((16,), jnp.float32)

        def scat(i, carry):
            idxv = didx[pl.ds(i * 16, 16)]
            plsc.addupdate_scatter(hist, [idxv], ones16)
            return carry

        lax.fori_loop(0, _EW // 16, scat, 0)
        pltpu.sync_copy(hist, out.at[wid])

    return deg


_NB = 3
_NG = _NCHUNK // _NB
_NTAIL = _NCHUNK - _NG * _NB


@functools.lru_cache(maxsize=None)
def _make_sc_agg(D, tc_tiling=True):
    params = None if tc_tiling else pltpu.CompilerParams(
        use_tc_tiling_on_sc=False)
    @functools.partial(
        pl.kernel,
        out_type=jax.ShapeDtypeStruct((_NC, _NPAD, D), jnp.float32),
        mesh=_mesh(),
        compiler_params=params,
        scratch_types=(
            [pltpu.VMEM((_CH,), jnp.int32)] * _NB
            + [pltpu.VMEM((_CH,), jnp.int32)] * _NB
            + [pltpu.VMEM((_CH, D), jnp.float32)] * _NB
            + [pltpu.VMEM_SHARED((_NPAD, D), jnp.float32)]
            + [pltpu.SemaphoreType.DMA] * (2 * _NB)
        ),
    )
    def agg(hs, srci, dsti, zeros_hbm, out, *refs):
        sidxs = refs[:_NB]
        didxs = refs[_NB:2 * _NB]
        rowss = refs[2 * _NB:3 * _NB]
        acc = refs[3 * _NB]
        sems = refs[3 * _NB + 1:3 * _NB + 1 + _NB]
        ssems = refs[3 * _NB + 1 + _NB:]
        c = lax.axis_index("c")
        s = lax.axis_index("s")
        wid = s * _NC + c
        pltpu.sync_copy(zeros_hbm, acc.at[pl.ds(s * _RPT, _RPT)])
        plsc.subcore_barrier()
        ebase = wid * _EW

        def group(g, carry):
            descs = []
            for b in range(_NB):
                off = ebase + (g * _NB + b) * _CH
                pltpu.sync_copy(srci.at[pl.ds(off, _CH)], sidxs[b])
                pltpu.sync_copy(dsti.at[pl.ds(off, _CH)], didxs[b])
                descs.append(pltpu.async_copy(hs.at[sidxs[b]], rowss[b],
                                              sems[b]))
            sdescs = []
            for b in range(_NB):
                descs[b].wait()
                sdescs.append(pltpu.async_copy(rowss[b], acc.at[didxs[b]],
                                               ssems[b], add=True))
            for b in range(_NB):
                sdescs[b].wait()
            return carry

        lax.fori_loop(0, _NG, group, 0)
        for t in range(_NTAIL):
            off = ebase + (_NG * _NB + t) * _CH
            pltpu.sync_copy(srci.at[pl.ds(off, _CH)], sidxs[0])
            pltpu.sync_copy(dsti.at[pl.ds(off, _CH)], didxs[0])
            pltpu.async_copy(hs.at[sidxs[0]], rowss[0], sems[0]).wait()
            pltpu.sync_copy(rowss[0], acc.at[didxs[0]], add=True)
        plsc.subcore_barrier()
        pltpu.sync_copy(acc.at[pl.ds(s * _RPT, _RPT)],
                        out.at[c, pl.ds(s * _RPT, _RPT)])

    return jax.jit(agg)


def _dinv_from(degT_ref):
    deg = jnp.sum(degT_ref[0:_N, :], axis=1, keepdims=True) + 1.0
    return lax.rsqrt(deg)


def _tc_a_body(x_ref, w1_ref, degT_ref, hs1_ref):
    dinv = _dinv_from(degT_ref)
    h = jnp.dot(x_ref[...], w1_ref[...], preferred_element_type=jnp.float32,
                precision=lax.Precision.HIGHEST)
    hs1_ref[...] = h * dinv


def _tc_b_body(agg_ref, hs1_ref, degT_ref, b1_ref, g1_ref, be1_ref, w2_ref,
               hs2_ref):
    dinv = _dinv_from(degT_ref)
    aggsum = agg_ref[0, 0:_N, :] + agg_ref[1, 0:_N, :]
    t = dinv * (aggsum + hs1_ref[...]) + b1_ref[...]
    mu = jnp.mean(t, axis=0, keepdims=True)
    var = jnp.mean((t - mu) ** 2, axis=0, keepdims=True)
    tn = g1_ref[...] * (t - mu) * lax.rsqrt(var + _EPS) + be1_ref[...]
    h = jnp.maximum(tn, 0.0)
    h2 = jnp.dot(h, w2_ref[...], preferred_element_type=jnp.float32,
                 precision=lax.Precision.HIGHEST)
    hs2_ref[...] = jnp.concatenate(
        [h2 * dinv, jnp.zeros((_N, 64), jnp.float32)], axis=1)


def _tc_c_body(agg_ref, hs2_ref, degT_ref, b2_ref, g2_ref, be2_ref,
               bcol_ref, blane_ref, G1_ref, gb1_ref, G2_ref, gb2_ref,
               C1_ref, cb1_ref, C2_ref, cb2_ref, out_ref):
    dinv = _dinv_from(degT_ref)
    aggsum = agg_ref[0, 0:_N, 0:64] + agg_ref[1, 0:_N, 0:64]
    t = dinv * (aggsum + hs2_ref[0:_N, 0:64]) + b2_ref[...]
    mu = jnp.mean(t, axis=0, keepdims=True)
    var = jnp.mean((t - mu) ** 2, axis=0, keepdims=True)
    tn = g2_ref[...] * (t - mu) * lax.rsqrt(var + _EPS) + be2_ref[...]
    h = jnp.maximum(tn, 0.0)

    g_hidden = jnp.maximum(
        jnp.dot(h, G1_ref[...], preferred_element_type=jnp.float32,
                precision=lax.Precision.HIGHEST) + gb1_ref[...], 0.0)
    gate = jnp.dot(g_hidden, G2_ref[...], preferred_element_type=jnp.float32,
                   precision=lax.Precision.HIGHEST) + gb2_ref[...]

    iota_col = lax.broadcasted_iota(jnp.int32, (_N, _G), 1)
    maskf = (bcol_ref[...] == iota_col).astype(jnp.float32)
    iota_lane = lax.broadcasted_iota(jnp.int32, (_G, _N), 0)
    maskTf = (blane_ref[...] == iota_lane).astype(jnp.float32)

    neg = jnp.float32(-jnp.inf)
    gmax = jnp.max(jnp.where(maskf > 0.0, gate, neg), axis=0, keepdims=True)
    gmax = jnp.where(jnp.isfinite(gmax), gmax, 0.0)
    gmaxn = jnp.sum(maskf * gmax, axis=1, keepdims=True)
    e = jnp.exp(gate - gmaxn)
    denom = jnp.sum(maskf * e, axis=0, keepdims=True)
    denomn = jnp.sum(maskf * denom, axis=1, keepdims=True)
    alpha = e / (denomn + 1e-16)
    weighted = alpha * h
    pooled = jnp.dot(maskTf, weighted, preferred_element_type=jnp.float32,
                     precision=lax.Precision.HIGHEST)

    z = jnp.maximum(
        jnp.dot(pooled, C1_ref[...], preferred_element_type=jnp.float32,
                precision=lax.Precision.HIGHEST) + cb1_ref[...], 0.0)
    out_ref[...] = jnp.dot(z, C2_ref[...], preferred_element_type=jnp.float32,
                           precision=lax.Precision.HIGHEST) + cb2_ref[...]


def kernel(x, edge_index, batch, W1, b1, gamma1, beta1, W2, b2, gamma2, beta2,
           G1, gb1, G2, gb2, C1, cb1, C2, cb2):
    src = edge_index[0]
    dst = edge_index[1]

    degp = _make_sc_deg()(dst)
    degT = degp.T

    hs1 = pl.pallas_call(
        _tc_a_body,
        out_shape=jax.ShapeDtypeStruct((_N, 128), jnp.float32),
    )(x, W1, degT)

    z128 = jnp.zeros((_RPT, 128), jnp.float32)
    agg1 = _make_sc_agg(128)(hs1, src, dst, z128)

    hs2 = pl.pallas_call(
        _tc_b_body,
        out_shape=jax.ShapeDtypeStruct((_N, 128), jnp.float32),
    )(agg1, hs1, degT, b1.reshape(1, 128), gamma1.reshape(1, 128),
      beta1.reshape(1, 128), W2)

    agg2 = _make_sc_agg(128)(hs2, src, dst, z128)

    out = pl.pallas_call(
        _tc_c_body,
        out_shape=jax.ShapeDtypeStruct((_G, 2), jnp.float32),
    )(agg2, hs2, degT, b2.reshape(1, 64), gamma2.reshape(1, 64),
      beta2.reshape(1, 64), batch.reshape(_N, 1), batch.reshape(1, _N),
      G1, gb1.reshape(1, 32), G2, gb2.reshape(1, 1),
      C1, cb1.reshape(1, 32), C2, cb2.reshape(1, 2))
    return out


# speedup vs baseline: 1.3154x; 1.1807x over previous
import functools

import jax
import jax.numpy as jnp
from jax import lax
from jax.experimental import pallas as pl
from jax.experimental.pallas import tpu as pltpu
from jax.experimental.pallas import tpu_sc as plsc

_N = 10000
_E = 320000
_G = 64
_EPS = 1e-5

_NC = 2
_NS = 16
_NW = _NC * _NS
_NPAD = 10240
_EW = _E // _NW
_CH = 80
_NCHUNK = _EW // _CH
_RPT = _NPAD // _NS


def _mesh():
    return plsc.VectorSubcoreMesh(core_axis_name="c", subcore_axis_name="s",
                                  num_cores=_NC, num_subcores=_NS)


_FK = 25


@functools.lru_cache(maxsize=None)
def _make_sc_deg():
    @functools.partial(
        pl.kernel,
        out_type=jax.ShapeDtypeStruct((_NW, _NPAD), jnp.float32),
        mesh=_mesh(),
        compiler_params=pltpu.CompilerParams(needs_layout_passes=False),
        scratch_types=[
            pltpu.VMEM((_EW,), jnp.int32),
            pltpu.VMEM((_NPAD,), jnp.float32),
        ],
    )
    def deg(dsti, out, didx, hist):
        c = lax.axis_index("c")
        s = lax.axis_index("s")
        wid = s * _NC + c
        pltpu.sync_copy(dsti.at[pl.ds(wid * _EW, _EW)], didx)

        def zero(i, carry):
            hist[pl.ds(i * 16, 16)] = jnp.zeros((16,), jnp.float32)
            return carry

        lax.fori_loop(0, _NPAD // 16, zero, 0)
        ones16 = jnp.ones((16,), jnp.float32)

        def scat(i, carry):
            idxv = didx[pl.ds(i * 16, 16)]
            plsc.addupdate_scatter(hist, [idxv], ones16)
            return carry

        lax.fori_loop(0, _EW // 16, scat, 0)
        pltpu.sync_copy(hist, out.at[wid])

    return deg


_NB = 3
_NG = _NCHUNK // _NB
_NTAIL = _NCHUNK - _NG * _NB


@functools.lru_cache(maxsize=None)
def _make_sc_agg(D, tc_tiling=True):
    params = None if tc_tiling else pltpu.CompilerParams(
        use_tc_tiling_on_sc=False)
    @functools.partial(
        pl.kernel,
        out_type=jax.ShapeDtypeStruct((_NC, _NPAD, D), jnp.float32),
        mesh=_mesh(),
        compiler_params=params,
        scratch_types=(
            [pltpu.VMEM((2, _CH), jnp.int32)] * _NB
            + [pltpu.VMEM((_CH, D), jnp.float32)] * _NB
            + [pltpu.VMEM_SHARED((_NPAD, D), jnp.float32)]
            + [pltpu.SemaphoreType.DMA] * (2 * _NB)
        ),
    )
    def agg(hs, sdp, zeros_hbm, out, *refs):
        sdbufs = refs[:_NB]
        rowss = refs[_NB:2 * _NB]
        acc = refs[2 * _NB]
        sems = refs[2 * _NB + 1:2 * _NB + 1 + _NB]
        ssems = refs[2 * _NB + 1 + _NB:]
        c = lax.axis_index("c")
        s = lax.axis_index("s")
        wid = s * _NC + c
        pltpu.sync_copy(zeros_hbm, acc.at[pl.ds(s * _RPT, _RPT)])
        plsc.subcore_barrier()

        def chunk_idx(b, i):
            pltpu.sync_copy(sdp.at[wid, i], sdbufs[b])

        def group(g, carry):
            descs = []
            for b in range(_NB):
                chunk_idx(b, g * _NB + b)
                descs.append(pltpu.async_copy(hs.at[sdbufs[b].at[0]],
                                              rowss[b], sems[b]))
            sdescs = []
            for b in range(_NB):
                descs[b].wait()
                sdescs.append(pltpu.async_copy(rowss[b],
                                               acc.at[sdbufs[b].at[1]],
                                               ssems[b], add=True))
            for b in range(_NB):
                sdescs[b].wait()
            return carry

        lax.fori_loop(0, _NG, group, 0)
        for t in range(_NTAIL):
            chunk_idx(0, _NG * _NB + t)
            pltpu.async_copy(hs.at[sdbufs[0].at[0]], rowss[0], sems[0]).wait()
            pltpu.sync_copy(rowss[0], acc.at[sdbufs[0].at[1]], add=True)
        plsc.subcore_barrier()
        pltpu.sync_copy(acc.at[pl.ds(s * _RPT, _RPT)],
                        out.at[c, pl.ds(s * _RPT, _RPT)])

    return jax.jit(agg)


def _dinv_from(degT_ref):
    deg = jnp.sum(degT_ref[0:_N, :], axis=1, keepdims=True) + 1.0
    return lax.rsqrt(deg)


def _tc_a_body(x_ref, w1_ref, degT_ref, hs1_ref):
    dinv = _dinv_from(degT_ref)
    h = jnp.dot(x_ref[...], w1_ref[...], preferred_element_type=jnp.float32,
                precision=lax.Precision.HIGHEST)
    hs1_ref[...] = h * dinv


def _tc_b_body(agg_ref, hs1_ref, degT_ref, b1_ref, g1_ref, be1_ref, w2_ref,
               hs2_ref):
    dinv = _dinv_from(degT_ref)
    aggsum = agg_ref[0, 0:_N, :] + agg_ref[1, 0:_N, :]
    t = dinv * (aggsum + hs1_ref[...]) + b1_ref[...]
    mu = jnp.mean(t, axis=0, keepdims=True)
    var = jnp.mean((t - mu) ** 2, axis=0, keepdims=True)
    tn = g1_ref[...] * (t - mu) * lax.rsqrt(var + _EPS) + be1_ref[...]
    h = jnp.maximum(tn, 0.0)
    h2 = jnp.dot(h, w2_ref[...], preferred_element_type=jnp.float32,
                 precision=lax.Precision.HIGHEST)
    hs2_ref[...] = jnp.concatenate(
        [h2 * dinv, jnp.zeros((_N, 64), jnp.float32)], axis=1)


def _tc_c_body(agg_ref, hs2_ref, degT_ref, b2_ref, g2_ref, be2_ref,
               bcol_ref, blane_ref, G1_ref, gb1_ref, G2_ref, gb2_ref,
               C1_ref, cb1_ref, C2_ref, cb2_ref, out_ref):
    dinv = _dinv_from(degT_ref)
    aggsum = agg_ref[0, 0:_N, 0:64] + agg_ref[1, 0:_N, 0:64]
    t = dinv * (aggsum + hs2_ref[0:_N, 0:64]) + b2_ref[...]
    mu = jnp.mean(t, axis=0, keepdims=True)
    var = jnp.mean((t - mu) ** 2, axis=0, keepdims=True)
    tn = g2_ref[...] * (t - mu) * lax.rsqrt(var + _EPS) + be2_ref[...]
    h = jnp.maximum(tn, 0.0)

    g_hidden = jnp.maximum(
        jnp.dot(h, G1_ref[...], preferred_element_type=jnp.float32,
                precision=lax.Precision.HIGHEST) + gb1_ref[...], 0.0)
    gate = jnp.dot(g_hidden, G2_ref[...], preferred_element_type=jnp.float32,
                   precision=lax.Precision.HIGHEST) + gb2_ref[...]

    iota_col = lax.broadcasted_iota(jnp.int32, (_N, _G), 1)
    maskf = (bcol_ref[...] == iota_col).astype(jnp.float32)
    iota_lane = lax.broadcasted_iota(jnp.int32, (_G, _N), 0)
    maskTf = (blane_ref[...] == iota_lane).astype(jnp.float32)

    neg = jnp.float32(-jnp.inf)
    gmax = jnp.max(jnp.where(maskf > 0.0, gate, neg), axis=0, keepdims=True)
    gmax = jnp.where(jnp.isfinite(gmax), gmax, 0.0)
    gmaxn = jnp.sum(maskf * gmax, axis=1, keepdims=True)
    e = jnp.exp(gate - gmaxn)
    denom = jnp.sum(maskf * e, axis=0, keepdims=True)
    denomn = jnp.sum(maskf * denom, axis=1, keepdims=True)
    alpha = e / (denomn + 1e-16)
    weighted = alpha * h
    pooled = jnp.dot(maskTf, weighted, preferred_element_type=jnp.float32,
                     precision=lax.Precision.HIGHEST)

    z = jnp.maximum(
        jnp.dot(pooled, C1_ref[...], preferred_element_type=jnp.float32,
                precision=lax.Precision.HIGHEST) + cb1_ref[...], 0.0)
    out_ref[...] = jnp.dot(z, C2_ref[...], preferred_element_type=jnp.float32,
                           precision=lax.Precision.HIGHEST) + cb2_ref[...]


def kernel(x, edge_index, batch, W1, b1, gamma1, beta1, W2, b2, gamma2, beta2,
           G1, gb1, G2, gb2, C1, cb1, C2, cb2):
    src = edge_index[0]
    dst = edge_index[1]

    degp = _make_sc_deg()(dst)
    degT = degp.T

    sdp = jnp.stack([src.reshape(_NW, _NCHUNK, _CH),
                     dst.reshape(_NW, _NCHUNK, _CH)], axis=2)

    hs1 = pl.pallas_call(
        _tc_a_body,
        out_shape=jax.ShapeDtypeStruct((_N, 128), jnp.float32),
    )(x, W1, degT)

    z128 = jnp.zeros((_RPT, 128), jnp.float32)
    agg1 = _make_sc_agg(128)(hs1, sdp, z128)

    hs2 = pl.pallas_call(
        _tc_b_body,
        out_shape=jax.ShapeDtypeStruct((_N, 128), jnp.float32),
    )(agg1, hs1, degT, b1.reshape(1, 128), gamma1.reshape(1, 128),
      beta1.reshape(1, 128), W2)

    agg2 = _make_sc_agg(128)(hs2, sdp, z128)

    out = pl.pallas_call(
        _tc_c_body,
        out_shape=jax.ShapeDtypeStruct((_G, 2), jnp.float32),
    )(agg2, hs2, degT, b2.reshape(1, 64), gamma2.reshape(1, 64),
      beta2.reshape(1, 64), batch.reshape(_N, 1), batch.reshape(1, _N),
      G1, gb1.reshape(1, 32), G2, gb2.reshape(1, 1),
      C1, cb1.reshape(1, 32), C2, cb2.reshape(1, 2))
    return out


# speedup vs baseline: 1.4047x; 1.0679x over previous
import functools

import jax
import jax.numpy as jnp
from jax import lax
from jax.experimental import pallas as pl
from jax.experimental.pallas import tpu as pltpu
from jax.experimental.pallas import tpu_sc as plsc

_N = 10000
_E = 320000
_G = 64
_EPS = 1e-5

_NC = 2
_NS = 16
_NW = _NC * _NS
_NPAD = 10240
_EW = _E // _NW
_CH = 80
_NCHUNK = _EW // _CH
_RPT = _NPAD // _NS


def _mesh():
    return plsc.VectorSubcoreMesh(core_axis_name="c", subcore_axis_name="s",
                                  num_cores=_NC, num_subcores=_NS)


_FK = 25


@functools.lru_cache(maxsize=None)
def _make_sc_deg():
    @functools.partial(
        pl.kernel,
        out_type=jax.ShapeDtypeStruct((_NW, _NPAD), jnp.float32),
        mesh=_mesh(),
        compiler_params=pltpu.CompilerParams(needs_layout_passes=False),
        scratch_types=[
            pltpu.VMEM((_EW,), jnp.int32),
            pltpu.VMEM((_NPAD,), jnp.float32),
        ],
    )
    def deg(dsti, out, didx, hist):
        c = lax.axis_index("c")
        s = lax.axis_index("s")
        wid = s * _NC + c
        pltpu.sync_copy(dsti.at[pl.ds(wid * _EW, _EW)], didx)

        def zero(i, carry):
            hist[pl.ds(i * 16, 16)] = jnp.zeros((16,), jnp.float32)
            return carry

        lax.fori_loop(0, _NPAD // 16, zero, 0)
        ones16 = jnp.ones((16,), jnp.float32)

        def scat(i, carry):
            idxv = didx[pl.ds(i * 16, 16)]
            plsc.addupdate_scatter(hist, [idxv], ones16)
            return carry

        lax.fori_loop(0, _EW // 16, scat, 0)
        pltpu.sync_copy(hist, out.at[wid])

    return deg


_NB = 4
_NG = _NCHUNK // _NB
_NTAIL = _NCHUNK - _NG * _NB


@functools.lru_cache(maxsize=None)
def _make_sc_agg(D, tc_tiling=True):
    params = None if tc_tiling else pltpu.CompilerParams(
        use_tc_tiling_on_sc=False)
    @functools.partial(
        pl.kernel,
        out_type=jax.ShapeDtypeStruct((_NC, _NPAD, D), jnp.float32),
        mesh=_mesh(),
        compiler_params=params,
        scratch_types=(
            [pltpu.VMEM((2, _CH), jnp.int32)] * _NB
            + [pltpu.VMEM((_CH, D), jnp.float32)] * _NB
            + [pltpu.VMEM_SHARED((_NPAD, D), jnp.float32)]
            + [pltpu.SemaphoreType.DMA] * (2 * _NB)
        ),
    )
    def agg(hs, sdp, zeros_hbm, out, *refs):
        sdbufs = refs[:_NB]
        rowss = refs[_NB:2 * _NB]
        acc = refs[2 * _NB]
        sems = refs[2 * _NB + 1:2 * _NB + 1 + _NB]
        ssems = refs[2 * _NB + 1 + _NB:]
        c = lax.axis_index("c")
        s = lax.axis_index("s")
        wid = s * _NC + c
        pltpu.sync_copy(zeros_hbm, acc.at[pl.ds(s * _RPT, _RPT)])
        plsc.subcore_barrier()

        def chunk_idx(b, i):
            pltpu.sync_copy(sdp.at[wid, i], sdbufs[b])

        def group(g, carry):
            descs = []
            for b in range(_NB):
                chunk_idx(b, g * _NB + b)
                descs.append(pltpu.async_copy(hs.at[sdbufs[b].at[0]],
                                              rowss[b], sems[b]))
            sdescs = []
            for b in range(_NB):
                descs[b].wait()
                sdescs.append(pltpu.async_copy(rowss[b],
                                               acc.at[sdbufs[b].at[1]],
                                               ssems[b], add=True))
            for b in range(_NB):
                sdescs[b].wait()
            return carry

        lax.fori_loop(0, _NG, group, 0)
        for t in range(_NTAIL):
            chunk_idx(0, _NG * _NB + t)
            pltpu.async_copy(hs.at[sdbufs[0].at[0]], rowss[0], sems[0]).wait()
            pltpu.sync_copy(rowss[0], acc.at[sdbufs[0].at[1]], add=True)
        plsc.subcore_barrier()
        pltpu.sync_copy(acc.at[pl.ds(s * _RPT, _RPT)],
                        out.at[c, pl.ds(s * _RPT, _RPT)])

    return jax.jit(agg)


def _dinv_from(degT_ref):
    deg = jnp.sum(degT_ref[0:_N, :], axis=1, keepdims=True) + 1.0
    return lax.rsqrt(deg)


def _tc_a_body(x_ref, w1_ref, degT_ref, hs1_ref):
    dinv = _dinv_from(degT_ref)
    h = jnp.dot(x_ref[...], w1_ref[...], preferred_element_type=jnp.float32,
                precision=lax.Precision.HIGHEST)
    hs1_ref[...] = h * dinv


def _tc_b_body(agg_ref, hs1_ref, degT_ref, b1_ref, g1_ref, be1_ref, w2_ref,
               hs2_ref):
    dinv = _dinv_from(degT_ref)
    aggsum = agg_ref[0, 0:_N, :] + agg_ref[1, 0:_N, :]
    t = dinv * (aggsum + hs1_ref[...]) + b1_ref[...]
    mu = jnp.mean(t, axis=0, keepdims=True)
    var = jnp.mean((t - mu) ** 2, axis=0, keepdims=True)
    tn = g1_ref[...] * (t - mu) * lax.rsqrt(var + _EPS) + be1_ref[...]
    h = jnp.maximum(tn, 0.0)
    h2 = jnp.dot(h, w2_ref[...], preferred_element_type=jnp.float32,
                 precision=lax.Precision.HIGHEST)
    hs2_ref[...] = jnp.concatenate(
        [h2 * dinv, jnp.zeros((_N, 64), jnp.float32)], axis=1)


def _tc_c_body(agg_ref, hs2_ref, degT_ref, b2_ref, g2_ref, be2_ref,
               bcol_ref, blane_ref, G1_ref, gb1_ref, G2_ref, gb2_ref,
               C1_ref, cb1_ref, C2_ref, cb2_ref, out_ref):
    dinv = _dinv_from(degT_ref)
    aggsum = agg_ref[0, 0:_N, 0:64] + agg_ref[1, 0:_N, 0:64]
    t = dinv * (aggsum + hs2_ref[0:_N, 0:64]) + b2_ref[...]
    mu = jnp.mean(t, axis=0, keepdims=True)
    var = jnp.mean((t - mu) ** 2, axis=0, keepdims=True)
    tn = g2_ref[...] * (t - mu) * lax.rsqrt(var + _EPS) + be2_ref[...]
    h = jnp.maximum(tn, 0.0)

    g_hidden = jnp.maximum(
        jnp.dot(h, G1_ref[...], preferred_element_type=jnp.float32,
                precision=lax.Precision.HIGHEST) + gb1_ref[...], 0.0)
    gate = jnp.dot(g_hidden, G2_ref[...], preferred_element_type=jnp.float32,
                   precision=lax.Precision.HIGHEST) + gb2_ref[...]

    iota_col = lax.broadcasted_iota(jnp.int32, (_N, _G), 1)
    maskf = (bcol_ref[...] == iota_col).astype(jnp.float32)
    iota_lane = lax.broadcasted_iota(jnp.int32, (_G, _N), 0)
    maskTf = (blane_ref[...] == iota_lane).astype(jnp.float32)

    neg = jnp.float32(-jnp.inf)
    gmax = jnp.max(jnp.where(maskf > 0.0, gate, neg), axis=0, keepdims=True)
    gmax = jnp.where(jnp.isfinite(gmax), gmax, 0.0)
    gmaxn = jnp.sum(maskf * gmax, axis=1, keepdims=True)
    e = jnp.exp(gate - gmaxn)
    denom = jnp.sum(maskf * e, axis=0, keepdims=True)
    denomn = jnp.sum(maskf * denom, axis=1, keepdims=True)
    alpha = e / (denomn + 1e-16)
    weighted = alpha * h
    pooled = jnp.dot(maskTf, weighted, preferred_element_type=jnp.float32,
                     precision=lax.Precision.HIGHEST)

    z = jnp.maximum(
        jnp.dot(pooled, C1_ref[...], preferred_element_type=jnp.float32,
                precision=lax.Precision.HIGHEST) + cb1_ref[...], 0.0)
    out_ref[...] = jnp.dot(z, C2_ref[...], preferred_element_type=jnp.float32,
                           precision=lax.Precision.HIGHEST) + cb2_ref[...]


def kernel(x, edge_index, batch, W1, b1, gamma1, beta1, W2, b2, gamma2, beta2,
           G1, gb1, G2, gb2, C1, cb1, C2, cb2):
    src = edge_index[0]
    dst = edge_index[1]

    degp = _make_sc_deg()(dst)
    degT = degp.T

    sdp = jnp.stack([src.reshape(_NW, _NCHUNK, _CH),
                     dst.reshape(_NW, _NCHUNK, _CH)], axis=2)

    hs1 = pl.pallas_call(
        _tc_a_body,
        out_shape=jax.ShapeDtypeStruct((_N, 128), jnp.float32),
    )(x, W1, degT)

    z128 = jnp.zeros((_RPT, 128), jnp.float32)
    agg1 = _make_sc_agg(128)(hs1, sdp, z128)

    hs2 = pl.pallas_call(
        _tc_b_body,
        out_shape=jax.ShapeDtypeStruct((_N, 128), jnp.float32),
    )(agg1, hs1, degT, b1.reshape(1, 128), gamma1.reshape(1, 128),
      beta1.reshape(1, 128), W2)

    agg2 = _make_sc_agg(128)(hs2, sdp, z128)

    out = pl.pallas_call(
        _tc_c_body,
        out_shape=jax.ShapeDtypeStruct((_G, 2), jnp.float32),
    )(agg2, hs2, degT, b2.reshape(1, 64), gamma2.reshape(1, 64),
      beta2.reshape(1, 64), batch.reshape(_N, 1), batch.reshape(1, _N),
      G1, gb1.reshape(1, 32), G2, gb2.reshape(1, 1),
      C1, cb1.reshape(1, 32), C2, cb2.reshape(1, 2))
    return out
